# Initial kernel scaffold; baseline (speedup 1.0000x reference)
import jax
import jax.numpy as jnp
from jax.experimental import pallas as pl


def kernel(x, edge_index, W0, b0, g0, be0, W1, b1, g1, be1, W2, b2):
    raise NotImplementedError("write your pallas kernel here")


# speedup vs baseline: 3.8784x; 3.8784x over previous
import functools

import jax
import jax.numpy as jnp
from jax import lax
from jax.experimental import pallas as pl
from jax.experimental.pallas import tpu as pltpu
from jax.experimental.pallas import tpu_sc as plsc

N = 10000
E = 160000
D = 256
H = D // 2

NTILES = 16
CHUNK = 128
EPAD = 163840
NCHUNKS = EPAD // (NTILES * CHUNK)
DUMMY_DST = N

HALF0 = 5120
NACC = 5248
ACC_DUMMY = HALF0
INIT_ROWS = NACC // NTILES

NPADH = 10240
HIST_INIT_ROWS = NPADH // NTILES

BR = 1000
GRID = N // BR

_sc_mesh = plsc.VectorSubcoreMesh(core_axis_name="c", subcore_axis_name="s")


@functools.partial(
    pl.kernel,
    out_type=jax.ShapeDtypeStruct((NPADH, 16), jnp.float32),
    mesh=_sc_mesh,
    scratch_types=[
        pltpu.VMEM((NCHUNKS, CHUNK), jnp.int32),
        pltpu.VMEM((CHUNK, 16), jnp.float32),
        pltpu.VMEM_SHARED((NPADH, 16), jnp.float32),
    ],
)
def _deg_kernel(dst_hbm, deg_out, dst_v, ones_v, acc_sh):
    c = lax.axis_index("c")
    s = lax.axis_index("s")

    pltpu.sync_copy(dst_hbm.at[s], dst_v)

    one_col = jnp.where(lax.iota(jnp.int32, 16) == 0, 1.0, 0.0)

    def fill(i, _):
        ones_v[i, :] = one_col
        return 0
    lax.fori_loop(0, CHUNK, fill, 0)

    base = s * HIST_INIT_ROWS
    for k in range(HIST_INIT_ROWS // CHUNK):
        pltpu.sync_copy(ones_v, acc_sh.at[pl.ds(base + k * CHUNK, CHUNK)])
    plsc.subcore_barrier()

    def body(j, _):
        pltpu.sync_copy(ones_v, acc_sh.at[dst_v.at[j]], add=True)
        return 0
    lax.fori_loop(0, NCHUNKS, body, 0)
    plsc.subcore_barrier()

    half = NPADH // 2
    rows_per_tile = half // NTILES
    obase = c * half + s * rows_per_tile
    pltpu.sync_copy(acc_sh.at[pl.ds(obase, rows_per_tile)],
                    deg_out.at[pl.ds(obase, rows_per_tile)])


@functools.partial(
    pl.kernel,
    out_type=jax.ShapeDtypeStruct((2 * N, H), jnp.float32),
    mesh=_sc_mesh,
    scratch_types=[
        pltpu.VMEM((NCHUNKS, CHUNK), jnp.int32),
        pltpu.VMEM((NCHUNKS, CHUNK), jnp.int32),
        pltpu.VMEM((NCHUNKS, CHUNK), jnp.int32),
        pltpu.VMEM((2, CHUNK, H), jnp.float32),
        pltpu.VMEM_SHARED((NACC, H), jnp.float32),
        pltpu.SemaphoreType.DMA((2,)),
    ],
)
def _segsum_kernel(u_hbm, src_hbm, dst_hbm, s_out,
                   gidx_v, dst_v, dstl_v, rows_v, acc_sh, sems):
    c = lax.axis_index("c")
    s = lax.axis_index("s")

    pltpu.sync_copy(src_hbm.at[s], gidx_v)
    pltpu.sync_copy(dst_hbm.at[s], dst_v)

    shift = (c * N).astype(jnp.int32)

    def shift_row(j, _):
        def shift_vec(v, _):
            sl = pl.ds(v * 16, 16)
            gidx_v[j, sl] = gidx_v[j, sl] + shift
            return 0
        lax.fori_loop(0, CHUNK // 16, shift_vec, 0)
        return 0
    lax.fori_loop(0, NCHUNKS, shift_row, 0)

    def localize_dst(lo, hi):
        def loc_row(j, _):
            def loc_vec(v, _):
                sl = pl.ds(v * 16, 16)
                d = dst_v[j, sl]
                t = d - lo
                ok = jnp.logical_and(d >= lo, d < hi)
                dstl_v[j, sl] = jnp.where(ok, t, ACC_DUMMY)
                return 0
            lax.fori_loop(0, CHUNK // 16, loc_vec, 0)
            return 0
        lax.fori_loop(0, NCHUNKS, loc_row, 0)

    def zero_acc():
        def zrow(i, _):
            def zvec(v, _):
                rows_v[0, i, pl.ds(v * 16, 16)] = jnp.zeros((16,), jnp.float32)
                return 0
            lax.fori_loop(0, H // 16, zvec, 0)
            return 0
        lax.fori_loop(0, CHUNK, zrow, 0)
        base = s * INIT_ROWS
        nfull = INIT_ROWS // CHUNK
        rem = INIT_ROWS - nfull * CHUNK
        for k in range(nfull):
            pltpu.sync_copy(rows_v.at[0],
                            acc_sh.at[pl.ds(base + k * CHUNK, CHUNK)])
        pltpu.sync_copy(rows_v.at[0].at[pl.ds(0, rem)],
                        acc_sh.at[pl.ds(base + nfull * CHUNK, rem)])

    def run_pass():
        pltpu.async_copy(u_hbm.at[gidx_v.at[0]], rows_v.at[0], sems.at[0])

        def body(j, _):
            slot = j % 2
            pltpu.make_async_copy(u_hbm.at[gidx_v.at[j]], rows_v.at[slot],
                                  sems.at[slot]).wait()

            @pl.when(j + 1 < NCHUNKS)
            def _():
                pltpu.async_copy(u_hbm.at[gidx_v.at[j + 1]],
                                 rows_v.at[1 - slot], sems.at[1 - slot])

            pltpu.sync_copy(rows_v.at[slot], acc_sh.at[dstl_v.at[j]],
                            add=True)
            return 0
        lax.fori_loop(0, NCHUNKS, body, 0)
        plsc.subcore_barrier()

    zero_acc()
    localize_dst(jnp.int32(0), jnp.int32(HALF0))
    plsc.subcore_barrier()
    run_pass()
    p0_rows = HALF0 // NTILES
    obase = s * p0_rows
    pltpu.sync_copy(acc_sh.at[pl.ds(obase, p0_rows)],
                    s_out.at[pl.ds(c * N + obase, p0_rows)])

    plsc.subcore_barrier()
    zero_acc()
    localize_dst(jnp.int32(HALF0), jnp.int32(N))
    plsc.subcore_barrier()
    run_pass()
    p1_rows = 304

    @pl.when(s < NTILES - 1)
    def _():
        ob = s * p1_rows
        pltpu.sync_copy(acc_sh.at[pl.ds(ob, p1_rows)],
                        s_out.at[pl.ds(c * N + HALF0 + ob, p1_rows)])

    @pl.when(s == NTILES - 1)
    def _():
        last = (NTILES - 1) * p1_rows
        nlast = (N - HALF0) - last
        pltpu.sync_copy(acc_sh.at[pl.ds(last, nlast)],
                        s_out.at[pl.ds(c * N + HALF0 + last, nlast)])


def _matmul_scale_body(x_ref, w_ref, deg_ref, u_ref):
    dinv = lax.rsqrt(deg_ref[:, 0:1])
    h = jnp.dot(x_ref[...], w_ref[...], preferred_element_type=jnp.float32)
    u = h * dinv
    u_ref[0, :, :] = u[:, :H]
    u_ref[1, :, :] = u[:, H:]


def _tc_matmul_scale(x, w, deg):
    return pl.pallas_call(
        _matmul_scale_body,
        grid=(GRID,),
        in_specs=[
            pl.BlockSpec((BR, D), lambda i: (i, 0)),
            pl.BlockSpec((D, D), lambda i: (0, 0)),
            pl.BlockSpec((BR, 16), lambda i: (i, 0)),
        ],
        out_specs=pl.BlockSpec((2, BR, H), lambda i: (0, i, 0)),
        out_shape=jax.ShapeDtypeStruct((2, N, H), jnp.float32),
    )(x, w, deg)


def _combine_stats_body(s_ref, u_ref, deg_ref, b_ref, y_ref, st_ref, acc):
    i = pl.program_id(0)
    dinv = lax.rsqrt(deg_ref[:, 0:1])
    t = jnp.concatenate([s_ref[0] + u_ref[0], s_ref[1] + u_ref[1]], axis=1)
    y = dinv * t + b_ref[...]
    y_ref[...] = y

    @pl.when(i == 0)
    def _():
        acc[...] = jnp.zeros_like(acc)

    acc[0:1, :] += jnp.sum(y, axis=0, keepdims=True)
    acc[1:2, :] += jnp.sum(y * y, axis=0, keepdims=True)

    @pl.when(i == GRID - 1)
    def _():
        st_ref[...] = acc[...]


def _tc_combine_stats(s_flat, u, deg, b):
    return pl.pallas_call(
        _combine_stats_body,
        grid=(GRID,),
        in_specs=[
            pl.BlockSpec((2, BR, H), lambda i: (0, i, 0)),
            pl.BlockSpec((2, BR, H), lambda i: (0, i, 0)),
            pl.BlockSpec((BR, 16), lambda i: (i, 0)),
            pl.BlockSpec((1, D), lambda i: (0, 0)),
        ],
        out_specs=[
            pl.BlockSpec((BR, D), lambda i: (i, 0)),
            pl.BlockSpec((2, D), lambda i: (0, 0)),
        ],
        out_shape=[
            jax.ShapeDtypeStruct((N, D), jnp.float32),
            jax.ShapeDtypeStruct((2, D), jnp.float32),
        ],
        scratch_shapes=[pltpu.VMEM((2, D), jnp.float32)],
    )(s_flat.reshape(2, N, H), u, deg, b)


def _bn_relu_matmul_body(y_ref, st_ref, g_ref, be_ref, w_ref, deg_ref, u_ref):
    mu = st_ref[0:1, :] * (1.0 / N)
    var = st_ref[1:2, :] * (1.0 / N) - mu * mu
    alpha = g_ref[...] * lax.rsqrt(var + 1e-5)
    z = (y_ref[...] - mu) * alpha + be_ref[...]
    h = jnp.maximum(z, 0.0)
    dinv = lax.rsqrt(deg_ref[:, 0:1])
    u = jnp.dot(h, w_ref[...], preferred_element_type=jnp.float32) * dinv
    u_ref[0, :, :] = u[:, :H]
    u_ref[1, :, :] = u[:, H:]


def _tc_bn_relu_matmul(y, st, g, be, w, deg):
    return pl.pallas_call(
        _bn_relu_matmul_body,
        grid=(GRID,),
        in_specs=[
            pl.BlockSpec((BR, D), lambda i: (i, 0)),
            pl.BlockSpec((2, D), lambda i: (0, 0)),
            pl.BlockSpec((1, D), lambda i: (0, 0)),
            pl.BlockSpec((1, D), lambda i: (0, 0)),
            pl.BlockSpec((D, D), lambda i: (0, 0)),
            pl.BlockSpec((BR, 16), lambda i: (i, 0)),
        ],
        out_specs=pl.BlockSpec((2, BR, H), lambda i: (0, i, 0)),
        out_shape=jax.ShapeDtypeStruct((2, N, H), jnp.float32),
    )(y, st, g, be, w, deg)


def _final_combine_body(s_ref, u_ref, deg_ref, b_ref, y_ref):
    dinv = lax.rsqrt(deg_ref[:, 0:1])
    t = jnp.concatenate([s_ref[0] + u_ref[0], s_ref[1] + u_ref[1]], axis=1)
    y_ref[...] = dinv * t + b_ref[...]


def _tc_final_combine(s_flat, u, deg, b):
    return pl.pallas_call(
        _final_combine_body,
        grid=(GRID,),
        in_specs=[
            pl.BlockSpec((2, BR, H), lambda i: (0, i, 0)),
            pl.BlockSpec((2, BR, H), lambda i: (0, i, 0)),
            pl.BlockSpec((BR, 16), lambda i: (i, 0)),
            pl.BlockSpec((1, D), lambda i: (0, 0)),
        ],
        out_specs=pl.BlockSpec((BR, D), lambda i: (i, 0)),
        out_shape=jax.ShapeDtypeStruct((N, D), jnp.float32),
    )(s_flat.reshape(2, N, H), u, deg, b)


@jax.jit
def kernel(x, edge_index, W0, b0, g0, be0, W1, b1, g1, be1, W2, b2):
    src = edge_index[0]
    dst = edge_index[1]
    pad = EPAD - E
    src_p = jnp.concatenate(
        [src, jnp.zeros((pad,), jnp.int32)]).reshape(NTILES, NCHUNKS, CHUNK)
    dst_p = jnp.concatenate(
        [dst, jnp.full((pad,), DUMMY_DST, jnp.int32)]
    ).reshape(NTILES, NCHUNKS, CHUNK)

    deg = _deg_kernel(dst_p)
    deg = deg[:N]

    b0r = b0.reshape(1, D)
    b1r = b1.reshape(1, D)
    b2r = b2.reshape(1, D)
    g0r = g0.reshape(1, D)
    g1r = g1.reshape(1, D)
    be0r = be0.reshape(1, D)
    be1r = be1.reshape(1, D)

    u0 = _tc_matmul_scale(x, W0, deg)
    s0 = _segsum_kernel(u0.reshape(2 * N, H), src_p, dst_p)
    y0, st0 = _tc_combine_stats(s0, u0, deg, b0r)

    u1 = _tc_bn_relu_matmul(y0, st0, g0r, be0r, W1, deg)
    s1 = _segsum_kernel(u1.reshape(2 * N, H), src_p, dst_p)
    y1, st1 = _tc_combine_stats(s1, u1, deg, b1r)

    u2 = _tc_bn_relu_matmul(y1, st1, g1r, be1r, W2, deg)
    s2 = _segsum_kernel(u2.reshape(2 * N, H), src_p, dst_p)
    out = _tc_final_combine(s2, u2, deg, b2r)
    return out
